# baseline (device time: 26548 ns/iter reference)
import jax
import jax.numpy as jnp
from jax import lax
from jax.experimental import pallas as pl
from jax.experimental.pallas import tpu as pltpu

N_DEV = 32
N_TOK = 512
D_MODEL = 256
H = 512
N_EXPERTS = 64
E_LOCAL = N_EXPERTS // N_DEV
TOK_PER_DEV = N_TOK // N_DEV


def kernel(x, router_W, route_idx, expert_W):
    def body(x_ref, rw_ref, idx_ref, ew_ref, out_ref,
             send_buf, recv_buf, send_sems, recv_sems):
        me = lax.axis_index("i")

        scores = jnp.dot(x_ref[:, :], rw_ref[:, :],
                         preferred_element_type=jnp.float32)
        m = jnp.max(scores, axis=1, keepdims=True)
        p = jnp.exp(scores - m)
        idx0 = idx_ref[:, 0:1]
        idx1 = idx_ref[:, 1:2]
        iota = lax.broadcasted_iota(jnp.int32, (N_TOK, N_EXPERTS), 1)
        g0 = jnp.sum(jnp.where(iota == idx0, p, 0.0), axis=1, keepdims=True)
        g1 = jnp.sum(jnp.where(iota == idx1, p, 0.0), axis=1, keepdims=True)
        gs = g0 + g1
        w0 = g0 / gs
        w1 = g1 / gs

        x_v = x_ref[:, :]
        acc = jnp.zeros((N_TOK, H), jnp.float32)
        for le in range(E_LOCAL):
            ge = me * E_LOCAL + le
            wle = (jnp.where(idx0 == ge, w0, 0.0)
                   + jnp.where(idx1 == ge, w1, 0.0))
            xw = (wle * x_v).astype(jnp.bfloat16)
            acc = acc + jnp.dot(xw, ew_ref[le, :, :].astype(jnp.bfloat16),
                                preferred_element_type=jnp.float32)

        for d in range(N_DEV):
            send_buf[d, :, :] = acc[d * TOK_PER_DEV:(d + 1) * TOK_PER_DEV,
                                    :].astype(jnp.bfloat16)

        recv_buf[me] = send_buf[me]

        sends = []
        for off in range(1, N_DEV):
            d = lax.rem(me + off, N_DEV)
            rdma = pltpu.make_async_remote_copy(
                src_ref=send_buf.at[d],
                dst_ref=recv_buf.at[me],
                send_sem=send_sems.at[d],
                recv_sem=recv_sems.at[me],
                device_id=(d,),
                device_id_type=pl.DeviceIdType.MESH,
            )
            rdma.start()
            sends.append(rdma)

        for off in range(1, N_DEV):
            s = lax.rem(me - off + N_DEV, N_DEV)
            recv = pltpu.make_async_remote_copy(
                src_ref=send_buf.at[s],
                dst_ref=recv_buf.at[s],
                send_sem=send_sems.at[s],
                recv_sem=recv_sems.at[s],
                device_id=(s,),
                device_id_type=pl.DeviceIdType.MESH,
            )
            recv.wait_recv()

        out_ref[:, :] = jnp.sum(recv_buf[:, :, :].astype(jnp.float32), axis=0)

        for rdma in sends:
            rdma.wait_send()

    return pl.pallas_call(
        body,
        out_shape=jax.ShapeDtypeStruct((TOK_PER_DEV, H), jnp.float32),
        in_specs=[pl.BlockSpec(memory_space=pltpu.VMEM)] * 4,
        out_specs=pl.BlockSpec(memory_space=pltpu.VMEM),
        scratch_shapes=[
            pltpu.VMEM((N_DEV, TOK_PER_DEV, H), jnp.bfloat16),
            pltpu.VMEM((N_DEV, TOK_PER_DEV, H), jnp.bfloat16),
            pltpu.SemaphoreType.DMA((N_DEV,)),
            pltpu.SemaphoreType.DMA((N_DEV,)),
        ],
    )(x, router_W, route_idx, expert_W)


# device time: 19744 ns/iter; 1.3446x vs baseline; 1.3446x over previous
import jax
import jax.numpy as jnp
from jax import lax
from jax.experimental import pallas as pl
from jax.experimental.pallas import tpu as pltpu

N_DEV = 32
N_TOK = 512
D_MODEL = 256
H = 512
N_EXPERTS = 64
E_LOCAL = N_EXPERTS // N_DEV
TOK_PER_DEV = N_TOK // N_DEV


def kernel(x, router_W, route_idx, expert_W):
    def body(x_ref, rw_ref, idx_ref, ew_ref, out_ref,
             send_buf, recv_buf, send_sems, recv_sems):
        me = lax.axis_index("i")

        barrier_sem = pltpu.get_barrier_semaphore()
        for off in range(1, N_DEV):
            d = lax.rem(me + off, N_DEV)
            pl.semaphore_signal(barrier_sem, inc=1, device_id=(d,),
                                device_id_type=pl.DeviceIdType.MESH)

        scores = jnp.dot(x_ref[:, :], rw_ref[:, :],
                         preferred_element_type=jnp.float32)
        m = jnp.max(scores, axis=1, keepdims=True)
        p = jnp.exp(scores - m)
        idx0 = idx_ref[:, 0:1]
        idx1 = idx_ref[:, 1:2]
        iota = lax.broadcasted_iota(jnp.int32, (N_TOK, N_EXPERTS), 1)
        g0 = jnp.sum(jnp.where(iota == idx0, p, 0.0), axis=1, keepdims=True)
        g1 = jnp.sum(jnp.where(iota == idx1, p, 0.0), axis=1, keepdims=True)
        gs = g0 + g1
        w0 = g0 / gs
        w1 = g1 / gs

        x_v = x_ref[:, :]
        acc = jnp.zeros((N_TOK, H), jnp.float32)
        for le in range(E_LOCAL):
            ge = me * E_LOCAL + le
            wle = (jnp.where(idx0 == ge, w0, 0.0)
                   + jnp.where(idx1 == ge, w1, 0.0))
            xw = (wle * x_v).astype(jnp.bfloat16)
            acc = acc + jnp.dot(xw, ew_ref[le, :, :].astype(jnp.bfloat16),
                                preferred_element_type=jnp.float32)

        for d in range(N_DEV):
            send_buf[d, :, :] = acc[d * TOK_PER_DEV:(d + 1) * TOK_PER_DEV,
                                    :].astype(jnp.bfloat16)

        recv_buf[me] = send_buf[me]

        pl.semaphore_wait(barrier_sem, N_DEV - 1)

        sends = []
        for off in range(1, N_DEV):
            d = lax.rem(me + off, N_DEV)
            rdma = pltpu.make_async_remote_copy(
                src_ref=send_buf.at[d],
                dst_ref=recv_buf.at[me],
                send_sem=send_sems.at[d],
                recv_sem=recv_sems.at[me],
                device_id=(d,),
                device_id_type=pl.DeviceIdType.MESH,
            )
            rdma.start()
            sends.append(rdma)

        for off in range(1, N_DEV):
            s = lax.rem(me - off + N_DEV, N_DEV)
            recv = pltpu.make_async_remote_copy(
                src_ref=send_buf.at[s],
                dst_ref=recv_buf.at[s],
                send_sem=send_sems.at[s],
                recv_sem=recv_sems.at[s],
                device_id=(s,),
                device_id_type=pl.DeviceIdType.MESH,
            )
            recv.wait_recv()

        out_ref[:, :] = jnp.sum(recv_buf[:, :, :].astype(jnp.float32), axis=0)

        for rdma in sends:
            rdma.wait_send()

    return pl.pallas_call(
        body,
        out_shape=jax.ShapeDtypeStruct((TOK_PER_DEV, H), jnp.float32),
        in_specs=[pl.BlockSpec(memory_space=pltpu.VMEM)] * 4,
        out_specs=pl.BlockSpec(memory_space=pltpu.VMEM),
        scratch_shapes=[
            pltpu.VMEM((N_DEV, TOK_PER_DEV, H), jnp.bfloat16),
            pltpu.VMEM((N_DEV, TOK_PER_DEV, H), jnp.bfloat16),
            pltpu.SemaphoreType.DMA((N_DEV,)),
            pltpu.SemaphoreType.DMA((N_DEV,)),
        ],
        compiler_params=pltpu.CompilerParams(collective_id=0),
    )(x, router_W, route_idx, expert_W)
